# baseline (speedup 1.0000x reference)
import functools

import jax
import jax.numpy as jnp
from jax import lax
from jax.experimental import pallas as pl
from jax.experimental.pallas import tpu as pltpu
from jax.experimental.pallas import tpu_sc as plsc

N = 100000
D = 128
DO = 256
C = 400
NCHUNK = N // C
NW = 32
CPW = -(-NCHUNK // NW)


def _body(element_hbm, x_hbm, table_hbm, out_hbm, idx_v, emb_v, sem):
    wid = lax.axis_index("s") * 2 + lax.axis_index("c")
    for j in range(CPW):
        cid = wid + j * NW

        @pl.when(cid < NCHUNK)
        def _():
            base = cid * C
            pltpu.sync_copy(element_hbm.at[pl.ds(base, C)], idx_v)
            pltpu.async_copy(table_hbm.at[idx_v], emb_v, sem).wait()
            pltpu.sync_copy(x_hbm.at[pl.ds(base, C), :],
                            out_hbm.at[pl.ds(base, C), pl.ds(D, D)])
            pltpu.sync_copy(emb_v, out_hbm.at[pl.ds(base, C), pl.ds(0, D)])


@jax.jit
def _sc_embed_concat(element, x, embed_table):
    mesh = plsc.VectorSubcoreMesh(core_axis_name="c", subcore_axis_name="s")
    return pl.kernel(
        _body,
        out_type=jax.ShapeDtypeStruct((N, DO), jnp.float32),
        mesh=mesh,
        scratch_types=[
            pltpu.VMEM((C,), jnp.int32),
            pltpu.VMEM((C, D), jnp.float32),
            pltpu.SemaphoreType.DMA,
        ],
    )(element, x, embed_table)


def kernel(element, x, embed_table):
    return _sc_embed_concat(element.astype(jnp.int32), x, embed_table)


# speedup vs baseline: 7.5143x; 7.5143x over previous
import jax
import jax.numpy as jnp
from jax import lax
from jax.experimental import pallas as pl
from jax.experimental.pallas import tpu as pltpu
from jax.experimental.pallas import tpu_sc as plsc

N = 100000
D = 128
DO = 256
C = 200
NCHUNK = N // C
NW = 32
CPW = -(-NCHUNK // NW)


def _body(element_hbm, x_hbm, table_hbm, out_hbm,
          idx0, idx1, emb_v, x_v, sem_i, sem_g, sem_x, sem_w0, sem_w1):
    wid = lax.axis_index("s") * 2 + lax.axis_index("c")
    idx = (idx0, idx1)
    sem_w = (sem_w0, sem_w1)

    def idx_copy(j):
        base = (wid + j * NW) * C
        return pltpu.make_async_copy(
            element_hbm.at[pl.ds(base, C)], idx[j % 2], sem_i)

    def emb_write(j, base):
        return pltpu.make_async_copy(
            emb_v.at[j % 2], out_hbm.at[pl.ds(base, C), pl.ds(0, D)],
            sem_w[j % 2])

    def x_write(j, base):
        return pltpu.make_async_copy(
            x_v.at[j % 2], out_hbm.at[pl.ds(base, C), pl.ds(D, D)],
            sem_w[j % 2])

    @pl.when(wid < NCHUNK)
    def _():
        idx_copy(0).start()

    for j in range(CPW):
        b = j % 2
        cid = wid + j * NW

        if j + 1 < CPW:
            @pl.when(wid + (j + 1) * NW < NCHUNK)
            def _():
                idx_copy(j + 1).start()

        @pl.when(cid < NCHUNK)
        def _():
            base = cid * C
            if j >= 2:
                emb_write(j, base).wait()
                x_write(j, base).wait()
            idx_copy(j).wait()
            g = pltpu.make_async_copy(
                table_hbm.at[idx[b]], emb_v.at[b], sem_g)
            g.start()
            xr = pltpu.make_async_copy(
                x_hbm.at[pl.ds(base, C), :], x_v.at[b], sem_x)
            xr.start()
            g.wait()
            emb_write(j, base).start()
            xr.wait()
            x_write(j, base).start()

    for j in (CPW - 2, CPW - 1):
        if j >= 0:
            cid = wid + j * NW

            @pl.when(cid < NCHUNK)
            def _():
                base = cid * C
                emb_write(j, base).wait()
                x_write(j, base).wait()


@jax.jit
def _sc_embed_concat(element, x, embed_table):
    mesh = plsc.VectorSubcoreMesh(core_axis_name="c", subcore_axis_name="s")
    return pl.kernel(
        _body,
        out_type=jax.ShapeDtypeStruct((N, DO), jnp.float32),
        mesh=mesh,
        scratch_types=[
            pltpu.VMEM((C,), jnp.int32),
            pltpu.VMEM((C,), jnp.int32),
            pltpu.VMEM((2, C, D), jnp.float32),
            pltpu.VMEM((2, C, D), jnp.float32),
            pltpu.SemaphoreType.DMA,
            pltpu.SemaphoreType.DMA,
            pltpu.SemaphoreType.DMA,
            pltpu.SemaphoreType.DMA,
            pltpu.SemaphoreType.DMA,
        ],
    )(element, x, embed_table)


def kernel(element, x, embed_table):
    return _sc_embed_concat(element.astype(jnp.int32), x, embed_table)


# speedup vs baseline: 8.4759x; 1.1280x over previous
import jax
import jax.numpy as jnp
from jax import lax
from jax.experimental import pallas as pl
from jax.experimental.pallas import tpu as pltpu
from jax.experimental.pallas import tpu_sc as plsc

N = 100000
D = 128
DO = 256
C = 200
NCHUNK = N // C
NW = 32
CPW = -(-NCHUNK // NW)


def _body(element_hbm, x_hbm, table_hbm, out_hbm, idx_v, emb_v, x_v, sem):
    wid = lax.axis_index("s") * 2 + lax.axis_index("c")
    for j in range(CPW):
        cid = wid + j * NW

        @pl.when(cid < NCHUNK)
        def _():
            base = cid * C
            pltpu.sync_copy(element_hbm.at[pl.ds(base, C)], idx_v)
            pltpu.async_copy(table_hbm.at[idx_v], emb_v, sem).wait()
            pltpu.sync_copy(x_hbm.at[pl.ds(base, C), :], x_v)
            pltpu.sync_copy(emb_v, out_hbm.at[pl.ds(base, C), pl.ds(0, D)])
            pltpu.sync_copy(x_v, out_hbm.at[pl.ds(base, C), pl.ds(D, D)])


@jax.jit
def _sc_embed_concat(element, x, embed_table):
    mesh = plsc.VectorSubcoreMesh(core_axis_name="c", subcore_axis_name="s")
    return pl.kernel(
        _body,
        out_type=jax.ShapeDtypeStruct((N, DO), jnp.float32),
        mesh=mesh,
        scratch_types=[
            pltpu.VMEM((C,), jnp.int32),
            pltpu.VMEM((C, D), jnp.float32),
            pltpu.VMEM((C, D), jnp.float32),
            pltpu.SemaphoreType.DMA,
        ],
    )(element, x, embed_table)


def kernel(element, x, embed_table):
    return _sc_embed_concat(element.astype(jnp.int32), x, embed_table)


# speedup vs baseline: 16.5743x; 1.9555x over previous
import jax
import jax.numpy as jnp
from jax import lax
from jax.experimental import pallas as pl
from jax.experimental.pallas import tpu as pltpu
from jax.experimental.pallas import tpu_sc as plsc

N = 100000
D = 128
DO = 256
C = 400
NCHUNK = N // C
NW = 32
CPW = -(-NCHUNK // NW)


def _body(element_hbm, x_hbm, table_hbm, out_hbm, idx_v, emb_v, x_v, table_s, sem):
    wid = lax.axis_index("s") * 2 + lax.axis_index("c")
    sid = lax.axis_index("s")

    @pl.when(sid == 0)
    def _():
        pltpu.sync_copy(table_hbm, table_s)

    plsc.subcore_barrier()

    for j in range(CPW):
        cid = wid + j * NW

        @pl.when(cid < NCHUNK)
        def _():
            base = cid * C
            pltpu.sync_copy(element_hbm.at[pl.ds(base, C)], idx_v)
            pltpu.async_copy(table_s.at[idx_v], emb_v, sem).wait()
            pltpu.sync_copy(x_hbm.at[pl.ds(base, C), :], x_v)
            pltpu.sync_copy(emb_v, out_hbm.at[pl.ds(base, C), pl.ds(0, D)])
            pltpu.sync_copy(x_v, out_hbm.at[pl.ds(base, C), pl.ds(D, D)])


@jax.jit
def _sc_embed_concat(element, x, embed_table):
    mesh = plsc.VectorSubcoreMesh(core_axis_name="c", subcore_axis_name="s")
    return pl.kernel(
        _body,
        out_type=jax.ShapeDtypeStruct((N, DO), jnp.float32),
        mesh=mesh,
        scratch_types=[
            pltpu.VMEM((C,), jnp.int32),
            pltpu.VMEM((C, D), jnp.float32),
            pltpu.VMEM((C, D), jnp.float32),
            pltpu.VMEM_SHARED((100, D), jnp.float32),
            pltpu.SemaphoreType.DMA,
        ],
    )(element, x, embed_table)


def kernel(element, x, embed_table):
    return _sc_embed_concat(element.astype(jnp.int32), x, embed_table)


# speedup vs baseline: 20.8171x; 1.2560x over previous
import jax
import jax.numpy as jnp
from jax import lax
from jax.experimental import pallas as pl
from jax.experimental.pallas import tpu as pltpu
from jax.experimental.pallas import tpu_sc as plsc

N = 100000
D = 128
DO = 256
NE = 100
NW = 32
SPAN = 3128
C = 248
CHUNKS = [C] * (SPAN // C) + ([SPAN % C] if SPAN % C else [])
OFFS = [sum(CHUNKS[:i]) for i in range(len(CHUNKS))]


def _body(element_hbm, x_hbm, table_hbm, out_hbm,
          idx_v, emb_v, x_v, table_s, sem_g, sem_x, sem_w0, sem_w1):
    wid = lax.axis_index("s") * 2 + lax.axis_index("c")
    sid = lax.axis_index("s")
    sem_w = (sem_w0, sem_w1)

    @pl.when(sid == 0)
    def _():
        pltpu.sync_copy(table_hbm, table_s)

    base = jnp.minimum(wid * SPAN, N - SPAN)
    pltpu.sync_copy(element_hbm.at[pl.ds(base, SPAN)], idx_v)
    plsc.subcore_barrier()

    def emb_write(j):
        b, off, c = j % 2, OFFS[j], CHUNKS[j]
        return pltpu.make_async_copy(
            emb_v.at[b, pl.ds(0, c), :],
            out_hbm.at[pl.ds(base + off, c), pl.ds(0, D)], sem_w[b])

    def x_write(j):
        b, off, c = j % 2, OFFS[j], CHUNKS[j]
        return pltpu.make_async_copy(
            x_v.at[b, pl.ds(0, c), :],
            out_hbm.at[pl.ds(base + off, c), pl.ds(D, D)], sem_w[b])

    for j, (off, c) in enumerate(zip(OFFS, CHUNKS)):
        b = j % 2
        if j >= 2:
            emb_write(j - 2).wait()
            x_write(j - 2).wait()
        g = pltpu.make_async_copy(
            table_s.at[idx_v.at[pl.ds(off, c)]],
            emb_v.at[b, pl.ds(0, c), :], sem_g)
        g.start()
        xr = pltpu.make_async_copy(
            x_hbm.at[pl.ds(base + off, c), :],
            x_v.at[b, pl.ds(0, c), :], sem_x)
        xr.start()
        g.wait()
        emb_write(j).start()
        xr.wait()
        x_write(j).start()

    for j in (len(CHUNKS) - 2, len(CHUNKS) - 1):
        emb_write(j).wait()
        x_write(j).wait()


@jax.jit
def _sc_embed_concat(element, x, embed_table):
    mesh = plsc.VectorSubcoreMesh(core_axis_name="c", subcore_axis_name="s")
    return pl.kernel(
        _body,
        out_type=jax.ShapeDtypeStruct((N, DO), jnp.float32),
        mesh=mesh,
        scratch_types=[
            pltpu.VMEM((SPAN,), jnp.int32),
            pltpu.VMEM((2, C, D), jnp.float32),
            pltpu.VMEM((2, C, D), jnp.float32),
            pltpu.VMEM_SHARED((NE, D), jnp.float32),
            pltpu.SemaphoreType.DMA,
            pltpu.SemaphoreType.DMA,
            pltpu.SemaphoreType.DMA,
            pltpu.SemaphoreType.DMA,
        ],
    )(element, x, embed_table)


def kernel(element, x, embed_table):
    return _sc_embed_concat(element.astype(jnp.int32), x, embed_table)
